# baseline (speedup 1.0000x reference)
import functools

import jax
import jax.numpy as jnp
from jax import lax
from jax.experimental import pallas as pl
from jax.experimental.pallas import tpu as pltpu
from jax.experimental.pallas import tpu_sc as plsc

_N = 10000
_E = 320000
_DN = 128
_DE = 16
_H1 = 288
_SLAB = _H1 // 2
_CH = 32
_NCH = _E // _CH
_NC, _NS, _L = 2, 16, 16
_AM = 128
_RPS = _N // _NS


def _leaky(v):
    return jnp.where(v > 0, v, 0.01 * v)


def _g_body(x_ref, w_ref, b_ref, o_ref):
    o_ref[...] = (
        jnp.dot(x_ref[...], w_ref[0], preferred_element_type=jnp.float32)
        + b_ref[0]
    )


def _proj_g(x, W1a3, b1r):
    nb = 5
    blk = _N // nb
    return pl.pallas_call(
        _g_body,
        grid=(_NC, nb),
        in_specs=[
            pl.BlockSpec((blk, _DN), lambda c, i: (i, 0)),
            pl.BlockSpec((1, _DN, _SLAB), lambda c, i: (c, 0, 0)),
            pl.BlockSpec((1, 1, _SLAB), lambda c, i: (c, 0, 0)),
        ],
        out_specs=pl.BlockSpec((blk, _SLAB), lambda c, i: (c * nb + i, 0)),
        out_shape=jax.ShapeDtypeStruct((_NC * _N, _SLAB), jnp.float32),
    )(x, W1a3, b1r)


def _a_body(e_ref, w_ref, o_ref):
    o_ref[...] = jnp.dot(e_ref[...], w_ref[0], preferred_element_type=jnp.float32)


def _proj_a(edge_attr, W1bm3):
    nb = 80
    blk = _E // nb
    return pl.pallas_call(
        _a_body,
        grid=(_NC, nb),
        in_specs=[
            pl.BlockSpec((blk, _DE), lambda c, i: (i, 0)),
            pl.BlockSpec((1, _DE, _AM), lambda c, i: (c, 0, 0)),
        ],
        out_specs=pl.BlockSpec((blk, _AM), lambda c, i: (c * nb + i, 0)),
        out_shape=jax.ShapeDtypeStruct((_NC * _E, _AM), jnp.float32),
    )(edge_attr, W1bm3)


def _atail_body(e4_ref, w_ref, o_ref):
    o_ref[...] = jnp.dot(e4_ref[...], w_ref[...], preferred_element_type=jnp.float32)


def _proj_atail(E4, W4blk):
    nb = 20
    blk = (_E // 4) // nb
    return pl.pallas_call(
        _atail_body,
        grid=(nb,),
        in_specs=[
            pl.BlockSpec((blk, 4 * _DE), lambda i: (i, 0)),
            pl.BlockSpec((4 * _DE, _AM), lambda i: (0, 0)),
        ],
        out_specs=pl.BlockSpec((blk, _AM), lambda i: (i, 0)),
        out_shape=jax.ShapeDtypeStruct((_E // 4, _AM), jnp.float32),
    )(E4, W4blk)


_KPS = _NCH // _NS
_KMAIN = (_KPS - 1) // 4 * 4


def _sc_edge(row, col, g_all, a01, atail):
    mesh = plsc.VectorSubcoreMesh(core_axis_name="c", subcore_axis_name="s")

    @functools.partial(
        pl.kernel,
        out_type=[
            jax.ShapeDtypeStruct((_NC * _N, _SLAB), jnp.float32),
            jax.ShapeDtypeStruct((_NC * _NS, _N), jnp.int32),
        ],
        mesh=mesh,
        compiler_params=pltpu.CompilerParams(
            use_tc_tiling_on_sc=False, needs_layout_passes=False
        ),
        scratch_types=(
            [pltpu.VMEM((_CH,), jnp.int32)] * 8
            + [pltpu.VMEM((_CH, _SLAB), jnp.float32)] * 2
            + [pltpu.VMEM((_CH, _AM), jnp.float32)] * 2
            + [pltpu.VMEM((_CH // 4, _AM), jnp.float32)] * 2
            + [pltpu.VMEM((_N,), jnp.int32)]
            + [pltpu.VMEM_SHARED((_N, _SLAB), jnp.float32)]
            + [pltpu.SemaphoreType.DMA] * 16
        ),
    )
    def k(row_h, col_h, g_h, am_h, at_h, acc_out, cnt_out, *scr):
        rowb = scr[0:4]
        colb = scr[4:8]
        gb = scr[8:10]
        abm = scr[10:12]
        abt = scr[12:14]
        cntb = scr[14]
        acc = scr[15]
        gsem = scr[16:18]
        asem = scr[18:20]
        tsem = scr[20:22]
        ssem = scr[22:24]
        irs = scr[24:28]
        ics = scr[28:32]

        c = lax.axis_index("c")
        s = lax.axis_index("s")
        roff = c * _N
        kbase = s * _KPS

        def e_of(kk):
            return (kbase + kk) * _CH

        def zrow(i, _):
            for j in range(_SLAB // _L):
                gb[0][i, pl.ds(j * _L, _L)] = jnp.zeros((_L,), jnp.float32)
            return 0
        lax.fori_loop(0, _CH, zrow, 0)
        base = s * _RPS
        nfull = _RPS // _CH
        for kb in range(nfull):
            pltpu.sync_copy(gb[0], acc.at[pl.ds(base + kb * _CH, _CH)])
        rem = _RPS - nfull * _CH
        pltpu.sync_copy(gb[0].at[pl.ds(0, rem)],
                        acc.at[pl.ds(base + nfull * _CH, rem)])

        def zc(i, _):
            cntb[pl.ds(i * _L, _L)] = jnp.zeros((_L,), jnp.int32)
            return 0
        lax.fori_loop(0, _N // _L, zc, 0)
        plsc.subcore_barrier()

        ones = jnp.ones((_L,), jnp.int32)

        def idx_issue(kk, slot):
            pltpu.async_copy(row_h.at[pl.ds(e_of(kk), _CH)], rowb[slot],
                             irs[slot])
            pltpu.async_copy(col_h.at[pl.ds(e_of(kk), _CH)], colb[slot],
                             ics[slot])

        def idx_wait(kk, slot):
            pltpu.make_async_copy(row_h.at[pl.ds(e_of(kk), _CH)], rowb[slot],
                                  irs[slot]).wait()
            pltpu.make_async_copy(col_h.at[pl.ds(e_of(kk), _CH)], colb[slot],
                                  ics[slot]).wait()

        def offset_rows(slot):
            for j in range(_CH // _L):
                rowb[slot][pl.ds(j * _L, _L)] = (
                    rowb[slot][pl.ds(j * _L, _L)] + roff)

        def ga_issue(kk, p, slot):
            pltpu.async_copy(g_h.at[rowb[slot]], gb[p], gsem[p])
            pltpu.async_copy(am_h.at[pl.ds(c * _E + e_of(kk), _CH)], abm[p],
                             asem[p])
            pltpu.async_copy(at_h.at[pl.ds(e_of(kk) // 4, _CH // 4)], abt[p],
                             tsem[p])

        def ga_wait(kk, p, slot):
            pltpu.make_async_copy(g_h.at[rowb[slot]], gb[p], gsem[p]).wait()
            pltpu.make_async_copy(am_h.at[pl.ds(c * _E + e_of(kk), _CH)],
                                  abm[p], asem[p]).wait()
            pltpu.make_async_copy(at_h.at[pl.ds(e_of(kk) // 4, _CH // 4)],
                                  abt[p], tsem[p]).wait()

        def scat_issue(p, slot):
            pltpu.async_copy(gb[p], acc.at[colb[slot]], ssem[p], add=True)

        def scat_wait(p, slot):
            pltpu.make_async_copy(gb[p], acc.at[colb[slot]], ssem[p]).wait()

        def compute(p):
            def rowfn(i, _):
                tcol = (i % 4) * 32 + 16 * c
                for j in range(_SLAB // _L):
                    if j < _AM // _L:
                        av = abm[p][i, pl.ds(j * _L, _L)]
                    else:
                        av = abt[p][i // 4, pl.ds(tcol, _L)]
                    v = gb[p][i, pl.ds(j * _L, _L)] + av
                    gb[p][i, pl.ds(j * _L, _L)] = jnp.maximum(v, v * 0.01)
                return 0
            lax.fori_loop(0, _CH, rowfn, 0)

        def count(slot):
            for j in range(_CH // _L):
                plsc.addupdate_scatter(
                    cntb, [colb[slot][pl.ds(j * _L, _L)]], ones)

        idx_issue(0, 0)
        idx_issue(1, 1)
        idx_wait(0, 0)
        offset_rows(0)
        ga_issue(0, 0, 0)

        def group(outer, _):
            for b in range(4):
                kk = outer * 4 + b
                p = b % 2
                q = 1 - p
                sl = b
                sl1 = (b + 1) % 4
                sl2 = (b + 2) % 4
                @pl.when(kk >= 1)
                def _():
                    scat_wait(q, (b + 3) % 4)
                idx_wait(kk + 1, sl1)
                offset_rows(sl1)
                ga_issue(kk + 1, q, sl1)
                @pl.when(kk + 2 <= _KPS - 1)
                def _():
                    idx_issue(kk + 2, sl2)
                ga_wait(kk, p, sl)
                compute(p)
                scat_issue(p, sl)
                count(sl)
            return 0
        lax.fori_loop(0, _KMAIN // 4, group, 0)

        kk = _KPS - 1
        scat_wait(1, 3)
        ga_wait(kk, 0, 0)
        compute(0)
        scat_issue(0, 0)
        count(0)
        scat_wait(0, 0)

        plsc.subcore_barrier()
        pltpu.sync_copy(acc.at[pl.ds(base, _RPS)],
                        acc_out.at[pl.ds(c * _N + base, _RPS)])
        wid = s * _NC + c
        pltpu.sync_copy(cntb, cnt_out.at[wid])

    return k(row, col, g_all, a01, atail)


def _final_body(x_ref, a0_ref, a1_ref, cnt_ref, w2_ref, b2_ref, w3_ref, b3_ref,
                w4_ref, b4_ref, o_ref):
    cnt = jnp.sum(cnt_ref[...], axis=1).astype(jnp.float32) * 0.5
    inv = 1.0 / jnp.maximum(cnt, 1.0)
    mask = (cnt > 0).astype(jnp.float32)
    hm0 = a0_ref[...] * inv[:, None]
    hm1 = a1_ref[...] * inv[:, None]
    w2 = w2_ref[...]
    mean = (
        jnp.dot(hm0, w2[:_SLAB], preferred_element_type=jnp.float32)
        + jnp.dot(hm1, w2[_SLAB:], preferred_element_type=jnp.float32)
        + mask[:, None] * b2_ref[...]
    )
    w3 = w3_ref[...]
    t = _leaky(
        jnp.dot(x_ref[...], w3[:_DN], preferred_element_type=jnp.float32)
        + jnp.dot(mean, w3[_DN:], preferred_element_type=jnp.float32)
        + b3_ref[...]
    )
    o_ref[...] = (
        jnp.dot(t, w4_ref[...], preferred_element_type=jnp.float32) + b4_ref[...]
    )


def _final(x, acc_all, cnt_all, W2, b2r, W3, b3r, W4, b4r):
    nb = 5
    blk = _N // nb
    h2 = 2 * (_SLAB + _DN)
    return pl.pallas_call(
        _final_body,
        grid=(nb,),
        in_specs=[
            pl.BlockSpec((blk, _DN), lambda i: (i, 0)),
            pl.BlockSpec((blk, _SLAB), lambda i: (i, 0)),
            pl.BlockSpec((blk, _SLAB), lambda i: (nb + i, 0)),
            pl.BlockSpec((blk, _NC * _NS), lambda i: (i, 0)),
            pl.BlockSpec((_H1, _SLAB), lambda i: (0, 0)),
            pl.BlockSpec((1, _SLAB), lambda i: (0, 0)),
            pl.BlockSpec((_SLAB + _DN, h2), lambda i: (0, 0)),
            pl.BlockSpec((1, h2), lambda i: (0, 0)),
            pl.BlockSpec((h2, _DN), lambda i: (0, 0)),
            pl.BlockSpec((1, _DN), lambda i: (0, 0)),
        ],
        out_specs=pl.BlockSpec((blk, _DN), lambda i: (i, 0)),
        out_shape=jax.ShapeDtypeStruct((_N, _DN), jnp.float32),
    )(x, acc_all, acc_all, cnt_all.T, W2, b2r, W3, b3r, W4, b4r)


def kernel(x, edge_index, edge_attr, u, batch, W1, b1, W2, b2, W3, b3, W4, b4):
    row = edge_index[0]
    col = edge_index[1]
    W1a3 = W1[:_DN].reshape(_DN, _NC, _SLAB).transpose(1, 0, 2)
    W1b = W1[_DN:]
    W1bm3 = jnp.stack([W1b[:, :_AM], W1b[:, _SLAB:_SLAB + _AM]])
    Wtail = jnp.concatenate(
        [W1b[:, _AM:_SLAB], W1b[:, _SLAB + _AM:]], axis=1)
    W4blk = jnp.kron(jnp.eye(4, dtype=W1.dtype), Wtail)
    E4 = edge_attr.reshape(_E // 4, 4 * _DE)
    b1r = b1.reshape(_NC, 1, _SLAB)
    g_all = _proj_g(x, W1a3, b1r)
    a01 = _proj_a(edge_attr, W1bm3)
    atail = _proj_atail(E4, W4blk)
    acc_all, cnt_all = _sc_edge(row, col, g_all, a01, atail)
    return _final(x, acc_all, cnt_all, W2, b2.reshape(1, -1), W3,
                  b3.reshape(1, -1), W4, b4.reshape(1, -1))


# speedup vs baseline: 1.6061x; 1.6061x over previous
import functools

import jax
import jax.numpy as jnp
from jax import lax
from jax.experimental import pallas as pl
from jax.experimental.pallas import tpu as pltpu
from jax.experimental.pallas import tpu_sc as plsc

_N = 10000
_E = 320000
_DN = 128
_DE = 16
_H1 = 288
_SLAB = _H1 // 2
_CH = 32
_NCH = _E // _CH
_NC, _NS, _L = 2, 16, 16
_AM = 128
_RPS = _N // _NS


def _leaky(v):
    return jnp.where(v > 0, v, 0.01 * v)


def _g_body(x_ref, w_ref, b_ref, o_ref):
    o_ref[...] = (
        jnp.dot(x_ref[...], w_ref[0], preferred_element_type=jnp.float32)
        + b_ref[0]
    )


def _proj_g(x, W1a3, b1r):
    nb = 5
    blk = _N // nb
    return pl.pallas_call(
        _g_body,
        grid=(_NC, nb),
        in_specs=[
            pl.BlockSpec((blk, _DN), lambda c, i: (i, 0)),
            pl.BlockSpec((1, _DN, _SLAB), lambda c, i: (c, 0, 0)),
            pl.BlockSpec((1, 1, _SLAB), lambda c, i: (c, 0, 0)),
        ],
        out_specs=pl.BlockSpec((blk, _SLAB), lambda c, i: (c * nb + i, 0)),
        out_shape=jax.ShapeDtypeStruct((_NC * _N, _SLAB), jnp.float32),
    )(x, W1a3, b1r)


def _a_body(e_ref, w_ref, o_ref):
    o_ref[...] = jnp.dot(e_ref[...], w_ref[0], preferred_element_type=jnp.float32)


def _proj_a(edge_attr, W1bm3):
    nb = 80
    blk = _E // nb
    return pl.pallas_call(
        _a_body,
        grid=(_NC, nb),
        in_specs=[
            pl.BlockSpec((blk, _DE), lambda c, i: (i, 0)),
            pl.BlockSpec((1, _DE, _AM), lambda c, i: (c, 0, 0)),
        ],
        out_specs=pl.BlockSpec((blk, _AM), lambda c, i: (c * nb + i, 0)),
        out_shape=jax.ShapeDtypeStruct((_NC * _E, _AM), jnp.float32),
    )(edge_attr, W1bm3)


def _atail_body(e8_ref, w_ref, o_ref):
    o_ref[...] = jnp.dot(e8_ref[...], w_ref[0], preferred_element_type=jnp.float32)


def _proj_atail(E8, W8blk):
    nb = 20
    blk = (_E // 8) // nb
    return pl.pallas_call(
        _atail_body,
        grid=(_NC, nb),
        in_specs=[
            pl.BlockSpec((blk, 8 * _DE), lambda c, i: (i, 0)),
            pl.BlockSpec((1, 8 * _DE, _AM), lambda c, i: (c, 0, 0)),
        ],
        out_specs=pl.BlockSpec((blk, _AM), lambda c, i: (c * nb + i, 0)),
        out_shape=jax.ShapeDtypeStruct((_NC * (_E // 8), _AM), jnp.float32),
    )(E8, W8blk)


_KPS = _NCH // _NS
_KMAIN = (_KPS - 1) // 4 * 4


def _sc_edge(row, col, g_all, a01, atail):
    mesh = plsc.VectorSubcoreMesh(core_axis_name="c", subcore_axis_name="s")

    @functools.partial(
        pl.kernel,
        out_type=[
            jax.ShapeDtypeStruct((_NC * _N, _SLAB), jnp.float32),
            jax.ShapeDtypeStruct((_NC * _NS, _N), jnp.int32),
        ],
        mesh=mesh,
        compiler_params=pltpu.CompilerParams(
            use_tc_tiling_on_sc=False, needs_layout_passes=False
        ),
        scratch_types=(
            [pltpu.VMEM((_CH,), jnp.int32)] * 8
            + [pltpu.VMEM((_CH, _SLAB), jnp.float32)] * 2
            + [pltpu.VMEM((_CH, _SLAB), jnp.float32)] * 2
            + [pltpu.VMEM((_N,), jnp.int32)]
            + [pltpu.VMEM_SHARED((_N, _SLAB), jnp.float32)]
            + [pltpu.SemaphoreType.DMA] * 16
        ),
    )
    def k(row_h, col_h, g_h, am_h, at_h, acc_out, cnt_out, *scr):
        rowb = scr[0:4]
        colb = scr[4:8]
        gb = scr[8:10]
        ab = scr[10:12]
        cntb = scr[12]
        acc = scr[13]
        gsem = scr[14:16]
        asem = scr[16:18]
        tsem = scr[18:20]
        ssem = scr[20:22]
        irs = scr[22:26]
        ics = scr[26:30]

        c = lax.axis_index("c")
        s = lax.axis_index("s")
        roff = c * _N
        kbase = s * _KPS

        def e_of(kk):
            return (kbase + kk) * _CH

        def zrow(i, _):
            for j in range(_SLAB // _L):
                gb[0][i, pl.ds(j * _L, _L)] = jnp.zeros((_L,), jnp.float32)
            return 0
        lax.fori_loop(0, _CH, zrow, 0)
        base = s * _RPS
        nfull = _RPS // _CH
        for kb in range(nfull):
            pltpu.sync_copy(gb[0], acc.at[pl.ds(base + kb * _CH, _CH)])
        rem = _RPS - nfull * _CH
        pltpu.sync_copy(gb[0].at[pl.ds(0, rem)],
                        acc.at[pl.ds(base + nfull * _CH, rem)])

        def zc(i, _):
            cntb[pl.ds(i * _L, _L)] = jnp.zeros((_L,), jnp.int32)
            return 0
        lax.fori_loop(0, _N // _L, zc, 0)
        plsc.subcore_barrier()

        ones = jnp.ones((_L,), jnp.int32)

        def idx_issue(kk, slot):
            pltpu.async_copy(row_h.at[pl.ds(e_of(kk), _CH)], rowb[slot],
                             irs[slot])
            pltpu.async_copy(col_h.at[pl.ds(e_of(kk), _CH)], colb[slot],
                             ics[slot])

        def idx_wait(kk, slot):
            pltpu.make_async_copy(row_h.at[pl.ds(e_of(kk), _CH)], rowb[slot],
                                  irs[slot]).wait()
            pltpu.make_async_copy(col_h.at[pl.ds(e_of(kk), _CH)], colb[slot],
                                  ics[slot]).wait()

        def offset_rows(slot):
            for j in range(_CH // _L):
                rowb[slot][pl.ds(j * _L, _L)] = (
                    rowb[slot][pl.ds(j * _L, _L)] + roff)

        def ga_issue(kk, p, slot):
            pltpu.async_copy(g_h.at[rowb[slot]], gb[p], gsem[p])
            pltpu.async_copy(am_h.at[pl.ds(c * _E + e_of(kk), _CH)],
                             ab[p].at[:, pl.ds(0, _AM)], asem[p])
            pltpu.async_copy(at_h.at[pl.ds(c * _E + e_of(kk), _CH)],
                             ab[p].at[:, pl.ds(_AM, _DE)], tsem[p])

        def ga_wait(kk, p, slot):
            pltpu.make_async_copy(g_h.at[rowb[slot]], gb[p], gsem[p]).wait()
            pltpu.make_async_copy(am_h.at[pl.ds(c * _E + e_of(kk), _CH)],
                                  ab[p].at[:, pl.ds(0, _AM)], asem[p]).wait()
            pltpu.make_async_copy(at_h.at[pl.ds(c * _E + e_of(kk), _CH)],
                                  ab[p].at[:, pl.ds(_AM, _DE)], tsem[p]).wait()

        def scat_issue(p, slot):
            pltpu.async_copy(gb[p], acc.at[colb[slot]], ssem[p], add=True)

        def scat_wait(p, slot):
            pltpu.make_async_copy(gb[p], acc.at[colb[slot]], ssem[p]).wait()

        def compute(p):
            def rowfn(i, _):
                for j in range(_SLAB // _L):
                    v = (gb[p][i, pl.ds(j * _L, _L)]
                         + ab[p][i, pl.ds(j * _L, _L)])
                    gb[p][i, pl.ds(j * _L, _L)] = jnp.maximum(v, v * 0.01)
                return 0
            lax.fori_loop(0, _CH, rowfn, 0)

        def count(slot):
            for j in range(_CH // _L):
                plsc.addupdate_scatter(
                    cntb, [colb[slot][pl.ds(j * _L, _L)]], ones)

        idx_issue(0, 0)
        idx_issue(1, 1)
        idx_wait(0, 0)
        offset_rows(0)
        ga_issue(0, 0, 0)

        def group(outer, _):
            for b in range(4):
                kk = outer * 4 + b
                p = b % 2
                q = 1 - p
                sl = b
                sl1 = (b + 1) % 4
                sl2 = (b + 2) % 4
                @pl.when(kk >= 1)
                def _():
                    scat_wait(q, (b + 3) % 4)
                idx_wait(kk + 1, sl1)
                offset_rows(sl1)
                ga_issue(kk + 1, q, sl1)
                @pl.when(kk + 2 <= _KPS - 1)
                def _():
                    idx_issue(kk + 2, sl2)
                ga_wait(kk, p, sl)
                compute(p)
                scat_issue(p, sl)
                count(sl)
            return 0
        lax.fori_loop(0, _KMAIN // 4, group, 0)

        kk = _KPS - 1
        scat_wait(1, 3)
        ga_wait(kk, 0, 0)
        compute(0)
        scat_issue(0, 0)
        count(0)
        scat_wait(0, 0)

        plsc.subcore_barrier()
        pltpu.sync_copy(acc.at[pl.ds(base, _RPS)],
                        acc_out.at[pl.ds(c * _N + base, _RPS)])
        wid = s * _NC + c
        pltpu.sync_copy(cntb, cnt_out.at[wid])

    return k(row, col, g_all, a01, atail)


def _final_body(x_ref, a0_ref, a1_ref, cnt_ref, w2_ref, b2_ref, w3_ref, b3_ref,
                w4_ref, b4_ref, o_ref):
    cnt = jnp.sum(cnt_ref[...], axis=1).astype(jnp.float32) * 0.5
    inv = 1.0 / jnp.maximum(cnt, 1.0)
    mask = (cnt > 0).astype(jnp.float32)
    hm0 = a0_ref[...] * inv[:, None]
    hm1 = a1_ref[...] * inv[:, None]
    w2 = w2_ref[...]
    mean = (
        jnp.dot(hm0, w2[:_SLAB], preferred_element_type=jnp.float32)
        + jnp.dot(hm1, w2[_SLAB:], preferred_element_type=jnp.float32)
        + mask[:, None] * b2_ref[...]
    )
    w3 = w3_ref[...]
    t = _leaky(
        jnp.dot(x_ref[...], w3[:_DN], preferred_element_type=jnp.float32)
        + jnp.dot(mean, w3[_DN:], preferred_element_type=jnp.float32)
        + b3_ref[...]
    )
    o_ref[...] = (
        jnp.dot(t, w4_ref[...], preferred_element_type=jnp.float32) + b4_ref[...]
    )


def _final(x, acc_all, cnt_all, W2, b2r, W3, b3r, W4, b4r):
    nb = 5
    blk = _N // nb
    h2 = 2 * (_SLAB + _DN)
    return pl.pallas_call(
        _final_body,
        grid=(nb,),
        in_specs=[
            pl.BlockSpec((blk, _DN), lambda i: (i, 0)),
            pl.BlockSpec((blk, _SLAB), lambda i: (i, 0)),
            pl.BlockSpec((blk, _SLAB), lambda i: (nb + i, 0)),
            pl.BlockSpec((blk, _NC * _NS), lambda i: (i, 0)),
            pl.BlockSpec((_H1, _SLAB), lambda i: (0, 0)),
            pl.BlockSpec((1, _SLAB), lambda i: (0, 0)),
            pl.BlockSpec((_SLAB + _DN, h2), lambda i: (0, 0)),
            pl.BlockSpec((1, h2), lambda i: (0, 0)),
            pl.BlockSpec((h2, _DN), lambda i: (0, 0)),
            pl.BlockSpec((1, _DN), lambda i: (0, 0)),
        ],
        out_specs=pl.BlockSpec((blk, _DN), lambda i: (i, 0)),
        out_shape=jax.ShapeDtypeStruct((_N, _DN), jnp.float32),
    )(x, acc_all, acc_all, cnt_all.T, W2, b2r, W3, b3r, W4, b4r)


def kernel(x, edge_index, edge_attr, u, batch, W1, b1, W2, b2, W3, b3, W4, b4):
    row = edge_index[0]
    col = edge_index[1]
    W1a3 = W1[:_DN].reshape(_DN, _NC, _SLAB).transpose(1, 0, 2)
    W1b = W1[_DN:]
    W1bm3 = jnp.stack([W1b[:, :_AM], W1b[:, _SLAB:_SLAB + _AM]])
    eye8 = jnp.eye(8, dtype=W1.dtype)
    W8blk = jnp.stack([
        jnp.kron(eye8, W1b[:, _AM:_SLAB]),
        jnp.kron(eye8, W1b[:, _SLAB + _AM:]),
    ])
    E8 = edge_attr.reshape(_E // 8, 8 * _DE)
    b1r = b1.reshape(_NC, 1, _SLAB)
    g_all = _proj_g(x, W1a3, b1r)
    a01 = _proj_a(edge_attr, W1bm3)
    atail = _proj_atail(E8, W8blk).reshape(_NC * _E, _DE)
    acc_all, cnt_all = _sc_edge(row, col, g_all, a01, atail)
    return _final(x, acc_all, cnt_all, W2, b2.reshape(1, -1), W3,
                  b3.reshape(1, -1), W4, b4.reshape(1, -1))
